# baseline (speedup 1.0000x reference)
import functools

import jax
import jax.numpy as jnp
from jax.experimental import pallas as pl
from jax.experimental.pallas import tpu as pltpu


def _swiglu_body(x_ref, wu_ref, wg_ref, wd_ref, o_ref):
    ff = pl.program_id(1)
    x = x_ref[...]
    up = jnp.dot(x, wu_ref[...], preferred_element_type=jnp.float32)
    gate = jnp.dot(x, wg_ref[...], preferred_element_type=jnp.float32)
    z = (up * gate * jax.lax.logistic(gate)).astype(jnp.bfloat16)
    part = jnp.dot(z, wd_ref[...], preferred_element_type=jnp.float32)

    @pl.when(ff == 0)
    def _init():
        o_ref[...] = part

    @pl.when(ff != 0)
    def _acc():
        o_ref[...] += part


@functools.partial(jax.jit, static_argnames=("bm", "bf"))
def _fused_swiglu(xf, wu, wg, wd, bm=512, bf=512):
    m, d_model = xf.shape
    d_ff = wu.shape[1]
    grid = (m // bm, d_ff // bf)
    return pl.pallas_call(
        _swiglu_body,
        grid=grid,
        in_specs=[
            pl.BlockSpec((bm, d_model), lambda i, j: (i, 0)),
            pl.BlockSpec((d_model, bf), lambda i, j: (0, j)),
            pl.BlockSpec((d_model, bf), lambda i, j: (0, j)),
            pl.BlockSpec((bf, d_model), lambda i, j: (j, 0)),
        ],
        out_specs=pl.BlockSpec((bm, d_model), lambda i, j: (i, 0)),
        out_shape=jax.ShapeDtypeStruct((m, d_model), jnp.float32),
        compiler_params=pltpu.CompilerParams(
            dimension_semantics=("parallel", "arbitrary"),
        ),
    )(xf, wu, wg, wd)


def kernel(x, Wupt, Wgatet, Wdownt):
    b, s, d_model = x.shape
    xf = x.reshape(b * s, d_model).astype(jnp.bfloat16)
    out = _fused_swiglu(
        xf,
        Wupt.astype(jnp.bfloat16),
        Wgatet.astype(jnp.bfloat16),
        Wdownt.astype(jnp.bfloat16),
    )
    return out.reshape(b, s, d_model)


# speedup vs baseline: 1.0163x; 1.0163x over previous
import functools

import jax
import jax.numpy as jnp
from jax.experimental import pallas as pl
from jax.experimental.pallas import tpu as pltpu


def _swiglu_body(nff, bf, x_ref, wu_ref, wg_ref, wd_ref, o_ref, z_ref):
    j = pl.program_id(1)
    jmod = jax.lax.rem(j, 2)

    x = x_ref[...]
    up = jnp.dot(x, wu_ref[...], preferred_element_type=jnp.float32)
    gate = jnp.dot(x, wg_ref[...], preferred_element_type=jnp.float32)
    z_ref[jmod] = (up * gate * jax.lax.logistic(gate)).astype(jnp.bfloat16)

    down = jnp.dot(z_ref[1 - jmod], wd_ref[...],
                   preferred_element_type=jnp.float32)
    prev = o_ref[...]
    o_ref[...] = (jnp.where(j > 1, prev, 0.0)
                  + jnp.where(j > 0, down, 0.0))


@functools.partial(jax.jit, static_argnames=("bm", "bf"))
def _fused_swiglu(xf, wu, wg, wd, bm=512, bf=1024):
    m, d_model = xf.shape
    d_ff = wu.shape[1]
    nff = d_ff // bf
    grid = (m // bm, nff + 1)
    return pl.pallas_call(
        functools.partial(_swiglu_body, nff, bf),
        grid=grid,
        in_specs=[
            pl.BlockSpec((bm, d_model), lambda i, j: (i, 0)),
            pl.BlockSpec((d_model, bf), lambda i, j: (0, jnp.minimum(j, nff - 1))),
            pl.BlockSpec((d_model, bf), lambda i, j: (0, jnp.minimum(j, nff - 1))),
            pl.BlockSpec((bf, d_model), lambda i, j: (jnp.maximum(j, 1) - 1, 0)),
        ],
        out_specs=pl.BlockSpec((bm, d_model), lambda i, j: (i, 0)),
        out_shape=jax.ShapeDtypeStruct((m, d_model), jnp.float32),
        scratch_shapes=[pltpu.VMEM((2, bm, bf), jnp.bfloat16)],
        compiler_params=pltpu.CompilerParams(
            dimension_semantics=("parallel", "arbitrary"),
        ),
    )(xf, wu, wg, wd)


def kernel(x, Wupt, Wgatet, Wdownt):
    b, s, d_model = x.shape
    xf = x.reshape(b * s, d_model).astype(jnp.bfloat16)
    out = _fused_swiglu(
        xf,
        Wupt.astype(jnp.bfloat16),
        Wgatet.astype(jnp.bfloat16),
        Wdownt.astype(jnp.bfloat16),
    )
    return out.reshape(b, s, d_model)


# speedup vs baseline: 1.0510x; 1.0341x over previous
import functools

import jax
import jax.numpy as jnp
from jax.experimental import pallas as pl
from jax.experimental.pallas import tpu as pltpu


def _swiglu_body(nff, bf, x_ref, wu_ref, wg_ref, wd_ref, o_ref, z_ref):
    j = pl.program_id(1)
    jmod = jax.lax.rem(j, 2)

    x = x_ref[...]
    up = jnp.dot(x, wu_ref[...], preferred_element_type=jnp.float32)
    gate = jnp.dot(x, wg_ref[...], preferred_element_type=jnp.float32)
    z_ref[jmod] = (up * gate * jax.lax.logistic(gate)).astype(jnp.bfloat16)

    down = jnp.dot(z_ref[1 - jmod], wd_ref[...],
                   preferred_element_type=jnp.float32)
    prev = o_ref[...]
    o_ref[...] = (jnp.where(j > 1, prev, 0.0)
                  + jnp.where(j > 0, down, 0.0))


@functools.partial(jax.jit, static_argnames=("bm", "bf"))
def _fused_swiglu(xf, wu, wg, wd, bm=1024, bf=512):
    m, d_model = xf.shape
    d_ff = wu.shape[1]
    nff = d_ff // bf
    grid = (m // bm, nff + 1)
    return pl.pallas_call(
        functools.partial(_swiglu_body, nff, bf),
        grid=grid,
        in_specs=[
            pl.BlockSpec((bm, d_model), lambda i, j: (i, 0)),
            pl.BlockSpec((d_model, bf), lambda i, j: (0, jnp.minimum(j, nff - 1))),
            pl.BlockSpec((d_model, bf), lambda i, j: (0, jnp.minimum(j, nff - 1))),
            pl.BlockSpec((bf, d_model), lambda i, j: (jnp.maximum(j, 1) - 1, 0)),
        ],
        out_specs=pl.BlockSpec((bm, d_model), lambda i, j: (i, 0)),
        out_shape=jax.ShapeDtypeStruct((m, d_model), jnp.float32),
        scratch_shapes=[pltpu.VMEM((2, bm, bf), jnp.bfloat16)],
        compiler_params=pltpu.CompilerParams(
            dimension_semantics=("parallel", "arbitrary"),
        ),
    )(xf, wu, wg, wd)


def kernel(x, Wupt, Wgatet, Wdownt):
    b, s, d_model = x.shape
    xf = x.reshape(b * s, d_model).astype(jnp.bfloat16)
    out = _fused_swiglu(
        xf,
        Wupt.astype(jnp.bfloat16),
        Wgatet.astype(jnp.bfloat16),
        Wdownt.astype(jnp.bfloat16),
    )
    return out.reshape(b, s, d_model)


# speedup vs baseline: 1.1085x; 1.0547x over previous
import functools

import jax
import jax.numpy as jnp
from jax.experimental import pallas as pl
from jax.experimental.pallas import tpu as pltpu


def _swiglu_body(nff, bf, x_ref, wu_ref, wg_ref, wd_ref, o_ref, z_ref):
    j = pl.program_id(1)

    @pl.when(j < nff)
    def _up_gate():
        x = x_ref[...]
        up = jnp.dot(x, wu_ref[...], preferred_element_type=jnp.float32)
        gate = jnp.dot(x, wg_ref[...], preferred_element_type=jnp.float32)
        z_ref[j] = (up * gate * jax.lax.logistic(gate)).astype(jnp.bfloat16)

    @pl.when(j >= nff)
    def _down():
        acc = jnp.dot(z_ref[0], wd_ref[:bf, :],
                      preferred_element_type=jnp.float32)
        for t in range(1, nff):
            acc += jnp.dot(z_ref[t], wd_ref[pl.ds(t * bf, bf), :],
                           preferred_element_type=jnp.float32)
        o_ref[...] = acc


@functools.partial(jax.jit, static_argnames=("bm", "bf", "bn"))
def _fused_swiglu(xf, wu, wg, wd, bm=1024, bf=512, bn=512):
    m, d_model = xf.shape
    d_ff = wu.shape[1]
    nff = d_ff // bf
    nout = d_model // bn
    grid = (m // bm, nff + nout)
    return pl.pallas_call(
        functools.partial(_swiglu_body, nff, bf),
        grid=grid,
        in_specs=[
            pl.BlockSpec((bm, d_model), lambda i, j: (i, 0)),
            pl.BlockSpec((d_model, bf),
                         lambda i, j: (0, jnp.minimum(j, nff - 1))),
            pl.BlockSpec((d_model, bf),
                         lambda i, j: (0, jnp.minimum(j, nff - 1))),
            pl.BlockSpec((d_ff, bn),
                         lambda i, j: (0, jnp.clip(j - nff, 0, nout - 1))),
        ],
        out_specs=pl.BlockSpec((bm, bn),
                               lambda i, j: (i, jnp.maximum(j - nff, 0))),
        out_shape=jax.ShapeDtypeStruct((m, d_model), jnp.float32),
        scratch_shapes=[pltpu.VMEM((nff, bm, bf), jnp.bfloat16)],
        compiler_params=pltpu.CompilerParams(
            dimension_semantics=("parallel", "arbitrary"),
        ),
    )(xf, wu, wg, wd)


def kernel(x, Wupt, Wgatet, Wdownt):
    b, s, d_model = x.shape
    xf = x.reshape(b * s, d_model).astype(jnp.bfloat16)
    out = _fused_swiglu(
        xf,
        Wupt.astype(jnp.bfloat16),
        Wgatet.astype(jnp.bfloat16),
        Wdownt.astype(jnp.bfloat16),
    )
    return out.reshape(b, s, d_model)


# speedup vs baseline: 1.1246x; 1.0145x over previous
import functools

import jax
import jax.numpy as jnp
from jax.experimental import pallas as pl
from jax.experimental.pallas import tpu as pltpu


def _upgate_body(x_ref, wu_ref, wg_ref, z_ref):
    x = x_ref[...]
    up = jnp.dot(x, wu_ref[...], preferred_element_type=jnp.float32)
    gate = jnp.dot(x, wg_ref[...], preferred_element_type=jnp.float32)
    z_ref[...] = (up * gate * jax.lax.logistic(gate)).astype(jnp.bfloat16)


def _down_body(z_ref, wd_ref, o_ref):
    o_ref[...] = jnp.dot(z_ref[...], wd_ref[...],
                         preferred_element_type=jnp.float32)


@functools.partial(jax.jit, static_argnames=("bm", "bnz", "bmd", "bn"))
def _fused_swiglu(xf, wu, wg, wd, bm=1024, bnz=1024, bmd=1024, bn=512):
    m, d_model = xf.shape
    d_ff = wu.shape[1]
    z = pl.pallas_call(
        _upgate_body,
        grid=(m // bm, d_ff // bnz),
        in_specs=[
            pl.BlockSpec((bm, d_model), lambda i, j: (i, 0)),
            pl.BlockSpec((d_model, bnz), lambda i, j: (0, j)),
            pl.BlockSpec((d_model, bnz), lambda i, j: (0, j)),
        ],
        out_specs=pl.BlockSpec((bm, bnz), lambda i, j: (i, j)),
        out_shape=jax.ShapeDtypeStruct((m, d_ff), jnp.bfloat16),
        compiler_params=pltpu.CompilerParams(
            dimension_semantics=("parallel", "arbitrary"),
        ),
    )(xf, wu, wg)
    return pl.pallas_call(
        _down_body,
        grid=(m // bmd, d_model // bn),
        in_specs=[
            pl.BlockSpec((bmd, d_ff), lambda i, j: (i, 0)),
            pl.BlockSpec((d_ff, bn), lambda i, j: (0, j)),
        ],
        out_specs=pl.BlockSpec((bmd, bn), lambda i, j: (i, j)),
        out_shape=jax.ShapeDtypeStruct((m, d_model), jnp.float32),
        compiler_params=pltpu.CompilerParams(
            dimension_semantics=("parallel", "arbitrary"),
        ),
    )(z, wd)


def kernel(x, Wupt, Wgatet, Wdownt):
    b, s, d_model = x.shape
    xf = x.reshape(b * s, d_model).astype(jnp.bfloat16)
    out = _fused_swiglu(
        xf,
        Wupt.astype(jnp.bfloat16),
        Wgatet.astype(jnp.bfloat16),
        Wdownt.astype(jnp.bfloat16),
    )
    return out.reshape(b, s, d_model)
